# Initial kernel scaffold; baseline (speedup 1.0000x reference)
import jax
import jax.numpy as jnp
from jax.experimental import pallas as pl


def kernel(x):
    raise NotImplementedError("write your pallas kernel here")


# speedup vs baseline: 1.1141x; 1.1141x over previous
import jax
import jax.numpy as jnp
from jax.experimental import pallas as pl

_NV = 8


def _vote_body(x_ref, o_ref):
    data = x_ref[0]
    nv, n = data.shape
    m = jnp.max(data)
    vgrid = jax.lax.broadcasted_iota(jnp.int32, (nv, n), 0)
    fgrid = jax.lax.broadcasted_iota(jnp.int32, (nv, n), 1)
    keys = jnp.where(data == m, fgrid * nv + vgrid, jnp.int32(2**31 - 1))
    vote = jnp.min(keys) % nv
    sel = (vgrid == vote)
    o_ref[0, 0, :] = jnp.sum(jnp.where(sel, data, 0.0), axis=0)


def _make_call(b, n, interpret=False):
    return pl.pallas_call(
        _vote_body,
        grid=(b,),
        in_specs=[pl.BlockSpec((1, _NV, n), lambda g: (g, 0, 0))],
        out_specs=pl.BlockSpec((1, 1, n), lambda g: (g, 0, 0)),
        out_shape=jax.ShapeDtypeStruct((b, 1, n), jnp.float32),
        interpret=interpret,
    )


def kernel(x):
    b = x.shape[0] // _NV
    xr = x.reshape(b, _NV, -1)
    n = xr.shape[-1]
    out = _make_call(b, n)(xr)
    return out.reshape(b, n)


# speedup vs baseline: 1.5150x; 1.3598x over previous
import jax
import jax.numpy as jnp
from jax.experimental import pallas as pl

_NV = 8


def _vote_body(x_ref, o_ref):
    data = x_ref[0]
    nv, n = data.shape
    rowmax = jnp.max(data, axis=1, keepdims=True)
    m = jnp.max(rowmax)
    ismax = rowmax == m
    count = jnp.sum(ismax.astype(jnp.int32))
    viota = jax.lax.broadcasted_iota(jnp.int32, (nv, 1), 0)
    vote_fast = jnp.min(jnp.where(ismax, viota, jnp.int32(nv)))

    def _tie_vote():
        vgrid = jax.lax.broadcasted_iota(jnp.int32, (nv, n), 0)
        fgrid = jax.lax.broadcasted_iota(jnp.int32, (nv, n), 1)
        keys = jnp.where(data == m, fgrid * nv + vgrid, jnp.int32(2**31 - 1))
        return jnp.min(keys) % nv

    vote = jax.lax.cond(count == 1, lambda: vote_fast, _tie_vote)
    o_ref[0, 0, :] = x_ref[0, vote, :]


def _make_call(b, n, interpret=False):
    return pl.pallas_call(
        _vote_body,
        grid=(b,),
        in_specs=[pl.BlockSpec((1, _NV, n), lambda g: (g, 0, 0))],
        out_specs=pl.BlockSpec((1, 1, n), lambda g: (g, 0, 0)),
        out_shape=jax.ShapeDtypeStruct((b, 1, n), jnp.float32),
        interpret=interpret,
    )


def kernel(x):
    b = x.shape[0] // _NV
    xr = x.reshape(b, _NV, -1)
    n = xr.shape[-1]
    out = _make_call(b, n)(xr)
    return out.reshape(b, n)


# speedup vs baseline: 2.4618x; 1.6249x over previous
import jax
import jax.numpy as jnp
from jax.experimental import pallas as pl

_NV = 8


def _vote_body(x_ref, o_ref):
    for g in range(_GB):
        _one_group(x_ref, o_ref, g)


def _one_group(x_ref, o_ref, g):
    data = x_ref[g]
    nv, n = data.shape
    rowmax = jnp.max(data, axis=1, keepdims=True)
    m = jnp.max(rowmax)
    ismax = rowmax == m
    count = jnp.sum(ismax.astype(jnp.int32))
    viota = jax.lax.broadcasted_iota(jnp.int32, (nv, 1), 0)
    vote_fast = jnp.min(jnp.where(ismax, viota, jnp.int32(nv)))

    def _tie_vote():
        vgrid = jax.lax.broadcasted_iota(jnp.int32, (nv, n), 0)
        fgrid = jax.lax.broadcasted_iota(jnp.int32, (nv, n), 1)
        keys = jnp.where(data == m, fgrid * nv + vgrid, jnp.int32(2**31 - 1))
        return jnp.min(keys) % nv

    vote = jax.lax.cond(count == 1, lambda: vote_fast, _tie_vote)
    o_ref[g, 0, :] = x_ref[g, vote, :]


_GB = 8


def _make_call(b, n, interpret=False):
    return pl.pallas_call(
        _vote_body,
        grid=(b // _GB,),
        in_specs=[pl.BlockSpec((_GB, _NV, n), lambda g: (g, 0, 0))],
        out_specs=pl.BlockSpec((_GB, 1, n), lambda g: (g, 0, 0)),
        out_shape=jax.ShapeDtypeStruct((b, 1, n), jnp.float32),
        interpret=interpret,
    )


def kernel(x):
    b = x.shape[0] // _NV
    xr = x.reshape(b, _NV, -1)
    n = xr.shape[-1]
    out = _make_call(b, n)(xr)
    return out.reshape(b, n)
